# baseline (speedup 1.0000x reference)
import functools

import jax
import jax.numpy as jnp
from jax import lax
from jax.experimental import pallas as pl
from jax.experimental.pallas import tpu as pltpu
from jax.experimental.pallas import tpu_sc as plsc

_N = 10000
_E = 320000
_F = 128

_NC = 2
_NS = 16
_NW = _NC * _NS
_EPW = _E // _NW
_K = 80
_CHUNKS = _EPW // _K
_PAIRS = _CHUNKS // 2
_WT = 10
_RPT = _N // _WT
_ZR = 40


def _sc_scatter_body(src_hbm, dst_hbm, h_hbm, out_hbm,
                     sidx0_v, sidx1_v, sidx2_v, sidx3_v, dstb_v,
                     rows0_v, rows1_v, rows2_v, rows3_v, agg_sh,
                     gsem0, gsem1, gsem2, gsem3, xsem0, xsem1, xsem2, xsem3,
                     dsem0, dsem1, dsem2, dsem3, ssem0, ssem1, ssem2, ssem3):
    cid = lax.axis_index("c")
    sid = lax.axis_index("s")
    wid = cid * _NS + sid

    rows = (rows0_v, rows1_v, rows2_v, rows3_v)
    sidx = (sidx0_v, sidx1_v, sidx2_v, sidx3_v)
    gsem = (gsem0, gsem1, gsem2, gsem3)
    xsem = (xsem0, xsem1, xsem2, xsem3)
    dsem = (dsem0, dsem1, dsem2, dsem3)
    ssem = (ssem0, ssem1, ssem2, ssem3)

    def _src_load(j, b):
        pltpu.async_copy(src_hbm.at[wid, j], sidx[b], xsem[b])

    def _src_wait(b):
        pltpu.make_async_copy(src_hbm.at[wid, 0], sidx[b], xsem[b]).wait()

    def _dst_load(j, b):
        pltpu.async_copy(dst_hbm.at[wid, j], dstb_v.at[b], dsem[b])

    def _dst_wait(b):
        pltpu.make_async_copy(dst_hbm.at[wid, 0], dstb_v.at[b],
                              dsem[b]).wait()

    def _gather(b):
        pltpu.async_copy(h_hbm.at[sidx[b]], rows[b], gsem[b])

    def _gather_wait(b):
        pltpu.make_async_copy(h_hbm.at[sidx[b]], rows[b], gsem[b]).wait()

    def _scatter(b):
        pltpu.async_copy(rows[b], agg_sh.at[dstb_v.at[b]], ssem[b],
                         add=True)

    def _scatter_wait(b):
        pltpu.make_async_copy(rows[b], agg_sh.at[dstb_v.at[b]],
                              ssem[b]).wait()

    @pl.when(sid < _WT)
    def _zfill():
        def _zrow(i, carry):
            for cbase in range(_F // 16):
                rows3_v[i, pl.ds(cbase * 16, 16)] = jnp.zeros((16,),
                                                              jnp.float32)
            return carry
        lax.fori_loop(0, _ZR, _zrow, 0)
        for r in range(_RPT // _ZR):
            pltpu.async_copy(rows3_v.at[pl.ds(0, _ZR)],
                             agg_sh.at[pl.ds(sid * _RPT + r * _ZR, _ZR)],
                             ssem0)

    _dst_load(0, 0)
    _dst_load(1, 1)
    _dst_load(2, 2)
    pltpu.sync_copy(src_hbm.at[wid, 0], sidx0_v)
    _gather(0)
    _src_load(1, 1)
    _src_load(2, 2)
    _src_load(3, 3)
    _src_wait(1)
    _gather(1)
    _src_wait(2)
    _gather(2)

    @pl.when(sid < _WT)
    def _zdrain():
        for r in range(_RPT // _ZR):
            pltpu.make_async_copy(rows3_v.at[pl.ds(0, _ZR)],
                                  agg_sh.at[pl.ds(0, _ZR)], ssem0).wait()
    plsc.subcore_barrier()

    def _quad(t, carry):
        j_base = 4 * t
        for o in range(4):
            j = j_base + o
            b = o
            b3 = (o + 3) % 4
            _gather_wait(b)

            @pl.when(j + 4 < _CHUNKS)
            def _():
                _src_load(j + 4, b)
            _dst_wait(b)
            _scatter(b)

            @pl.when(j >= 1)
            def _():
                _scatter_wait(b3)

            @pl.when(j + 3 < _CHUNKS)
            def _():
                _dst_load(j + 3, b3)
                _src_wait(b3)
                _gather(b3)
        return carry

    lax.fori_loop(0, _CHUNKS // 4, _quad, 0)
    _gather_wait(0)
    _dst_wait(0)
    _scatter(0)
    _scatter_wait(3)
    _scatter_wait(0)
    plsc.subcore_barrier()

    @pl.when(sid < _WT)
    def _writeout():
        pltpu.sync_copy(agg_sh.at[pl.ds(sid * _RPT, _RPT)],
                        out_hbm.at[cid, pl.ds(sid * _RPT, _RPT)])


_sc_scatter = functools.partial(
    pl.kernel,
    out_type=jax.ShapeDtypeStruct((_NC, _N, _F), jnp.float32),
    mesh=plsc.VectorSubcoreMesh(core_axis_name="c", subcore_axis_name="s",
                                num_cores=_NC, num_subcores=_NS),
    scratch_types=(
        [pltpu.VMEM((_K,), jnp.int32) for _ in range(4)]
        + [pltpu.VMEM((4, _K), jnp.int32)]
        + [pltpu.VMEM((_K, _F), jnp.float32) for _ in range(4)]
        + [pltpu.VMEM_SHARED((_N, _F), jnp.float32)]
        + [pltpu.SemaphoreType.DMA for _ in range(16)]
    ),
)(_sc_scatter_body)


def _tc_epilogue_body(h_ref, prev_ref, norm_ref, agg0_ref, agg1_ref,
                      wl_ref, wsk_ref, b_ref, out_ref):
    prev = prev_ref[...]
    sw = jax.nn.sigmoid(
        jnp.dot(prev, wsk_ref[...], preferred_element_type=jnp.float32)
        + b_ref[...])
    lm = jnp.dot(h_ref[...], wl_ref[...], preferred_element_type=jnp.float32)
    node = (agg0_ref[...] + agg1_ref[...]) * norm_ref[...] + lm
    out = sw * node + (1.0 - sw) * prev
    u = lax.bitcast_convert_type(out, jnp.uint32)
    lsb = (u >> 13) & jnp.uint32(1)
    u = (u + jnp.uint32(0x0FFF) + lsb) & jnp.uint32(0xFFFFE000)
    out_ref[...] = lax.bitcast_convert_type(u, jnp.float32)


def _tc_epilogue(h, prev_h, norm, agg0, agg1, wl, wsk, b):
    blk = 1000
    grid = (_N // blk,)
    row_spec = pl.BlockSpec((blk, _F), lambda i: (i, 0))
    full_spec = pl.BlockSpec((_F, _F), lambda i: (0, 0))
    return pl.pallas_call(
        _tc_epilogue_body,
        grid=grid,
        in_specs=[
            row_spec,
            row_spec,
            pl.BlockSpec((blk, 1), lambda i: (i, 0)),
            row_spec,
            row_spec,
            full_spec,
            full_spec,
            pl.BlockSpec((1, _F), lambda i: (0, 0)),
        ],
        out_specs=row_spec,
        out_shape=jax.ShapeDtypeStruct((_N, _F), jnp.float32),
    )(h, prev_h, norm, agg0, agg1, wl, wsk, b)


def kernel(h, norm, prev_h, loop_weight, skip_connect_weight,
           skip_connect_bias, edge_index):
    src = edge_index[0].reshape(_NW, _CHUNKS, _K)
    dst = edge_index[1].reshape(_NW, _CHUNKS, _K)
    parts = _sc_scatter(src, dst, h)
    return _tc_epilogue(h, prev_h, norm, parts[0], parts[1],
                        loop_weight, skip_connect_weight,
                        skip_connect_bias.reshape(1, _F))


# speedup vs baseline: 1.0132x; 1.0132x over previous
import functools

import jax
import jax.numpy as jnp
from jax import lax
from jax.experimental import pallas as pl
from jax.experimental.pallas import tpu as pltpu
from jax.experimental.pallas import tpu_sc as plsc

_N = 10000
_E = 320000
_F = 128

_NC = 2
_NS = 16
_NW = _NC * _NS
_EPW = _E // _NW
_K = 80
_CHUNKS = _EPW // _K
_PAIRS = _CHUNKS // 2
_WT = 10
_RPT = _N // _WT
_ZR = 40


def _sc_scatter_body(src_hbm, dst_hbm, h_hbm, out_hbm,
                     sidx0_v, sidx1_v, sidx2_v, dst_v,
                     rows0_v, rows1_v, rows2_v, agg_sh,
                     gsem0, gsem1, gsem2, isem0, isem1, isem2,
                     ssem0, ssem1, ssem2):
    cid = lax.axis_index("c")
    sid = lax.axis_index("s")
    wid = cid * _NS + sid

    rows = (rows0_v, rows1_v, rows2_v)
    sidx = (sidx0_v, sidx1_v, sidx2_v)
    gsem = (gsem0, gsem1, gsem2)
    isem = (isem0, isem1, isem2)
    ssem = (ssem0, ssem1, ssem2)

    def _idx_load(j, b):
        pltpu.async_copy(src_hbm.at[wid, j], sidx[b], isem[b])

    def _idx_wait(b):
        pltpu.make_async_copy(src_hbm.at[wid, 0], sidx[b], isem[b]).wait()

    def _gather(b):
        pltpu.async_copy(h_hbm.at[sidx[b]], rows[b], gsem[b])

    def _gather_wait(b):
        pltpu.make_async_copy(h_hbm.at[sidx[b]], rows[b], gsem[b]).wait()

    def _scatter(j, b):
        pltpu.async_copy(rows[b], agg_sh.at[dst_v.at[j]], ssem[b], add=True)

    def _scatter_wait(b):
        pltpu.make_async_copy(rows[b], agg_sh.at[dst_v.at[0]],
                              ssem[b]).wait()

    @pl.when(sid < _WT)
    def _zfill():
        def _zrow(i, carry):
            for cbase in range(_F // 16):
                rows2_v[i, pl.ds(cbase * 16, 16)] = jnp.zeros((16,),
                                                              jnp.float32)
            return carry
        lax.fori_loop(0, _ZR, _zrow, 0)
        for r in range(_RPT // _ZR):
            pltpu.async_copy(rows2_v.at[pl.ds(0, _ZR)],
                             agg_sh.at[pl.ds(sid * _RPT + r * _ZR, _ZR)],
                             ssem0)

    pltpu.sync_copy(dst_hbm.at[wid], dst_v)
    pltpu.sync_copy(src_hbm.at[wid, 0], sidx0_v)
    _gather(0)
    _idx_load(1, 1)
    _idx_load(2, 2)
    _idx_wait(1)
    _gather(1)

    @pl.when(sid < _WT)
    def _zdrain():
        for r in range(_RPT // _ZR):
            pltpu.make_async_copy(rows2_v.at[pl.ds(0, _ZR)],
                                  agg_sh.at[pl.ds(0, _ZR)], ssem0).wait()
    plsc.subcore_barrier()

    def _triple(t, carry):
        j_base = 3 * t
        for o in range(3):
            j = j_base + o
            b = o
            b2 = (o + 2) % 3
            _gather_wait(b)
            _scatter(j, b)

            @pl.when(j >= 1)
            def _():
                _scatter_wait(b2)
            _idx_wait(b2)
            _gather(b2)

            @pl.when(j + 3 < _CHUNKS)
            def _():
                _idx_load(j + 3, b)
        return carry

    lax.fori_loop(0, _CHUNKS // 3, _triple, 0)
    _gather_wait(0)
    _scatter(_CHUNKS - 2, 0)
    _gather_wait(1)
    _scatter(_CHUNKS - 1, 1)
    _scatter_wait(2)
    _scatter_wait(0)
    _scatter_wait(1)
    plsc.subcore_barrier()

    @pl.when(sid < _NS - 1)
    def _writeout():
        pltpu.sync_copy(agg_sh.at[pl.ds(sid * 624, 624)],
                        out_hbm.at[cid, pl.ds(sid * 624, 624)])

    @pl.when(sid == _NS - 1)
    def _writeout_last():
        pltpu.sync_copy(agg_sh.at[pl.ds((_NS - 1) * 624, _N - (_NS - 1) * 624)],
                        out_hbm.at[cid, pl.ds((_NS - 1) * 624,
                                              _N - (_NS - 1) * 624)])


_sc_scatter = functools.partial(
    pl.kernel,
    out_type=jax.ShapeDtypeStruct((_NC, _N, _F), jnp.float32),
    mesh=plsc.VectorSubcoreMesh(core_axis_name="c", subcore_axis_name="s",
                                num_cores=_NC, num_subcores=_NS),
    scratch_types=[
        pltpu.VMEM((_K,), jnp.int32),
        pltpu.VMEM((_K,), jnp.int32),
        pltpu.VMEM((_K,), jnp.int32),
        pltpu.VMEM((_CHUNKS, _K), jnp.int32),
        pltpu.VMEM((_K, _F), jnp.float32),
        pltpu.VMEM((_K, _F), jnp.float32),
        pltpu.VMEM((_K, _F), jnp.float32),
        pltpu.VMEM_SHARED((_N, _F), jnp.float32),
        pltpu.SemaphoreType.DMA,
        pltpu.SemaphoreType.DMA,
        pltpu.SemaphoreType.DMA,
        pltpu.SemaphoreType.DMA,
        pltpu.SemaphoreType.DMA,
        pltpu.SemaphoreType.DMA,
        pltpu.SemaphoreType.DMA,
        pltpu.SemaphoreType.DMA,
        pltpu.SemaphoreType.DMA,
    ],
)(_sc_scatter_body)


def _tc_epilogue_body(h_ref, prev_ref, norm_ref, agg0_ref, agg1_ref,
                      wl_ref, wsk_ref, b_ref, out_ref):
    prev = prev_ref[...]
    sw = jax.nn.sigmoid(
        jnp.dot(prev, wsk_ref[...], preferred_element_type=jnp.float32)
        + b_ref[...])
    lm = jnp.dot(h_ref[...], wl_ref[...], preferred_element_type=jnp.float32)
    node = (agg0_ref[...] + agg1_ref[...]) * norm_ref[...] + lm
    out = sw * node + (1.0 - sw) * prev
    u = lax.bitcast_convert_type(out, jnp.uint32)
    lsb = (u >> 13) & jnp.uint32(1)
    u = (u + jnp.uint32(0x0FFF) + lsb) & jnp.uint32(0xFFFFE000)
    out_ref[...] = lax.bitcast_convert_type(u, jnp.float32)


def _tc_epilogue(h, prev_h, norm, agg0, agg1, wl, wsk, b):
    blk = 1000
    grid = (_N // blk,)
    row_spec = pl.BlockSpec((blk, _F), lambda i: (i, 0))
    full_spec = pl.BlockSpec((_F, _F), lambda i: (0, 0))
    return pl.pallas_call(
        _tc_epilogue_body,
        grid=grid,
        in_specs=[
            row_spec,
            row_spec,
            pl.BlockSpec((blk, 1), lambda i: (i, 0)),
            row_spec,
            row_spec,
            full_spec,
            full_spec,
            pl.BlockSpec((1, _F), lambda i: (0, 0)),
        ],
        out_specs=row_spec,
        out_shape=jax.ShapeDtypeStruct((_N, _F), jnp.float32),
    )(h, prev_h, norm, agg0, agg1, wl, wsk, b)


def kernel(h, norm, prev_h, loop_weight, skip_connect_weight,
           skip_connect_bias, edge_index):
    src = edge_index[0].reshape(_NW, _CHUNKS, _K)
    dst = edge_index[1].reshape(_NW, _CHUNKS, _K)
    parts = _sc_scatter(src, dst, h)
    return _tc_epilogue(h, prev_h, norm, parts[0], parts[1],
                        loop_weight, skip_connect_weight,
                        skip_connect_bias.reshape(1, _F))


# speedup vs baseline: 1.0161x; 1.0029x over previous
import functools

import jax
import jax.numpy as jnp
from jax import lax
from jax.experimental import pallas as pl
from jax.experimental.pallas import tpu as pltpu
from jax.experimental.pallas import tpu_sc as plsc

_N = 10000
_E = 320000
_F = 128

_NC = 2
_NS = 16
_NW = _NC * _NS
_EPW = _E // _NW
_K = 80
_CHUNKS = _EPW // _K
_WT = 10
_RPT = _N // _WT
_ZR = 40


def _sc_scatter_body(src_hbm, dst_hbm, h_hbm, out_hbm,
                     sidx0_v, sidx1_v, sidx2_v, dst_v,
                     rows0_v, rows1_v, rows2_v, agg_sh,
                     gsem0, gsem1, gsem2, isem0, isem1, isem2,
                     ssem0, ssem1, ssem2):
    cid = lax.axis_index("c")
    sid = lax.axis_index("s")
    wid = cid * _NS + sid

    rows = (rows0_v, rows1_v, rows2_v)
    sidx = (sidx0_v, sidx1_v, sidx2_v)
    gsem = (gsem0, gsem1, gsem2)
    isem = (isem0, isem1, isem2)
    ssem = (ssem0, ssem1, ssem2)

    def _idx_load(j, b):
        pltpu.async_copy(src_hbm.at[wid, j], sidx[b], isem[b])

    def _idx_wait(b):
        pltpu.make_async_copy(src_hbm.at[wid, 0], sidx[b], isem[b]).wait()

    def _gather(b):
        pltpu.async_copy(h_hbm.at[sidx[b]], rows[b], gsem[b])

    def _gather_wait(b):
        pltpu.make_async_copy(h_hbm.at[sidx[b]], rows[b], gsem[b]).wait()

    def _scatter(j, b):
        pltpu.async_copy(rows[b], agg_sh.at[dst_v.at[j]], ssem[b], add=True)

    def _scatter_wait(b):
        pltpu.make_async_copy(rows[b], agg_sh.at[dst_v.at[0]],
                              ssem[b]).wait()

    @pl.when(sid < _WT)
    def _zfill():
        def _zrow(i, carry):
            for cbase in range(_F // 16):
                rows2_v[i, pl.ds(cbase * 16, 16)] = jnp.zeros((16,),
                                                              jnp.float32)
            return carry
        lax.fori_loop(0, _ZR, _zrow, 0)
        for r in range(_RPT // _ZR):
            pltpu.async_copy(rows2_v.at[pl.ds(0, _ZR)],
                             agg_sh.at[pl.ds(sid * _RPT + r * _ZR, _ZR)],
                             ssem0)

    pltpu.sync_copy(dst_hbm.at[wid], dst_v)
    pltpu.sync_copy(src_hbm.at[wid, 0], sidx0_v)
    _gather(0)
    _idx_load(1, 1)
    _idx_load(2, 2)
    _idx_wait(1)
    _gather(1)

    @pl.when(sid < _WT)
    def _zdrain():
        for r in range(_RPT // _ZR):
            pltpu.make_async_copy(rows2_v.at[pl.ds(0, _ZR)],
                                  agg_sh.at[pl.ds(0, _ZR)], ssem0).wait()
    plsc.subcore_barrier()

    def _triple(t, carry):
        j_base = 3 * t
        for o in range(3):
            j = j_base + o
            b = o
            b2 = (o + 2) % 3
            _gather_wait(b)
            _scatter(j, b)

            @pl.when(j >= 1)
            def _():
                _scatter_wait(b2)
            _idx_wait(b2)
            _gather(b2)

            @pl.when(j + 3 < _CHUNKS)
            def _():
                _idx_load(j + 3, b)
        return carry

    lax.fori_loop(0, _CHUNKS // 3, _triple, 0)
    _gather_wait(0)
    _scatter(_CHUNKS - 2, 0)
    _gather_wait(1)
    _scatter(_CHUNKS - 1, 1)
    _scatter_wait(2)
    _scatter_wait(0)
    _scatter_wait(1)
    plsc.subcore_barrier()

    @pl.when(sid < _NS - 1)
    def _writeout():
        pltpu.sync_copy(agg_sh.at[pl.ds(sid * 624, 624)],
                        out_hbm.at[cid, pl.ds(sid * 624, 624)])

    @pl.when(sid == _NS - 1)
    def _writeout_last():
        pltpu.sync_copy(agg_sh.at[pl.ds((_NS - 1) * 624, _N - (_NS - 1) * 624)],
                        out_hbm.at[cid, pl.ds((_NS - 1) * 624,
                                              _N - (_NS - 1) * 624)])


_sc_scatter = functools.partial(
    pl.kernel,
    out_type=jax.ShapeDtypeStruct((_NC, _N, _F), jnp.float32),
    mesh=plsc.VectorSubcoreMesh(core_axis_name="c", subcore_axis_name="s",
                                num_cores=_NC, num_subcores=_NS),
    scratch_types=[
        pltpu.VMEM((_K,), jnp.int32),
        pltpu.VMEM((_K,), jnp.int32),
        pltpu.VMEM((_K,), jnp.int32),
        pltpu.VMEM((_CHUNKS, _K), jnp.int32),
        pltpu.VMEM((_K, _F), jnp.float32),
        pltpu.VMEM((_K, _F), jnp.float32),
        pltpu.VMEM((_K, _F), jnp.float32),
        pltpu.VMEM_SHARED((_N, _F), jnp.float32),
        pltpu.SemaphoreType.DMA,
        pltpu.SemaphoreType.DMA,
        pltpu.SemaphoreType.DMA,
        pltpu.SemaphoreType.DMA,
        pltpu.SemaphoreType.DMA,
        pltpu.SemaphoreType.DMA,
        pltpu.SemaphoreType.DMA,
        pltpu.SemaphoreType.DMA,
        pltpu.SemaphoreType.DMA,
    ],
)(_sc_scatter_body)


def _tc_epilogue_body(h_ref, prev_ref, norm_ref, agg0_ref, agg1_ref,
                      wl_ref, wsk_ref, b_ref, out_ref):
    prev = prev_ref[...]
    sw = jax.nn.sigmoid(
        jnp.dot(prev, wsk_ref[...], preferred_element_type=jnp.float32)
        + b_ref[...])
    lm = jnp.dot(h_ref[...], wl_ref[...], preferred_element_type=jnp.float32)
    node = (agg0_ref[...] + agg1_ref[...]) * norm_ref[...] + lm
    out = sw * node + (1.0 - sw) * prev
    u = lax.bitcast_convert_type(out, jnp.uint32)
    lsb = (u >> 13) & jnp.uint32(1)
    u = (u + jnp.uint32(0x0FFF) + lsb) & jnp.uint32(0xFFFFE000)
    out_ref[...] = lax.bitcast_convert_type(u, jnp.float32)


def _tc_epilogue(h, prev_h, norm, agg0, agg1, wl, wsk, b):
    blk = 1000
    grid = (_N // blk,)
    row_spec = pl.BlockSpec((blk, _F), lambda i: (i, 0))
    full_spec = pl.BlockSpec((_F, _F), lambda i: (0, 0))
    return pl.pallas_call(
        _tc_epilogue_body,
        grid=grid,
        in_specs=[
            row_spec,
            row_spec,
            pl.BlockSpec((blk, 1), lambda i: (i, 0)),
            row_spec,
            row_spec,
            full_spec,
            full_spec,
            pl.BlockSpec((1, _F), lambda i: (0, 0)),
        ],
        out_specs=row_spec,
        out_shape=jax.ShapeDtypeStruct((_N, _F), jnp.float32),
    )(h, prev_h, norm, agg0, agg1, wl, wsk, b)


def kernel(h, norm, prev_h, loop_weight, skip_connect_weight,
           skip_connect_bias, edge_index):
    src = edge_index[0].reshape(_NW, _CHUNKS, _K)
    dst = edge_index[1].reshape(_NW, _CHUNKS, _K)
    parts = _sc_scatter(src, dst, h)
    return _tc_epilogue(h, prev_h, norm, parts[0], parts[1],
                        loop_weight, skip_connect_weight,
                        skip_connect_bias.reshape(1, _F))
